# baseline (speedup 1.0000x reference)
import functools

import jax
import jax.numpy as jnp
from jax import lax
from jax.experimental import pallas as pl
from jax.experimental.pallas import tpu as pltpu
from jax.experimental.pallas import tpu_sc as plsc

N = 10000
E = 160000
LATENT = 64
STEPS = 3

NC = 2
NS = 16
NW = NC * NS
CH = 128
E_PAD = ((E + NW * CH - 1) // (NW * CH)) * (NW * CH)
BLK_E = 2048
GRID_E = E_PAD // BLK_E
BLK_EE = 2000
BLK_N = 1000
GRID_N = N // BLK_N
ACC_ROWS = ((N + NS - 1) // NS + 1) * NS
ROWS_PER_TILE = ACC_ROWS // NS

_G_TOT_CH = 2 * E_PAD // CH
_G_NCH0 = 112
_G_NCH1 = (_G_TOT_CH - NS * _G_NCH0) // NS
_G_GRP = 4
_G_ROWS = _G_GRP * CH
_S_PER_W = E_PAD // NW
_S_NCH = _S_PER_W // CH
_S_GRP = 4
_S_NGRP = _S_NCH // _S_GRP
_S_ROWS = _S_GRP * CH

assert _G_NCH0 % _G_GRP == 0 and _G_NCH1 % _G_GRP == 0
assert (_G_NCH0 // _G_GRP) % 2 == 0 and (_G_NCH1 // _G_GRP) % 2 == 0


def _ln(h, g, beta):
    mu = jnp.mean(h, axis=-1, keepdims=True)
    var = jnp.mean((h - mu) ** 2, axis=-1, keepdims=True)
    return (h - mu) * jax.lax.rsqrt(var + 1e-5) * g + beta


def _relu(v):
    return jnp.maximum(v, 0.0)


def _bdot(a, w):
    return jnp.dot(a.astype(jnp.bfloat16), w.astype(jnp.bfloat16),
                   preferred_element_type=jnp.float32)


def _fdot(a, w):
    return jnp.dot(a, w, preferred_element_type=jnp.float32,
                   precision=jax.lax.Precision.HIGHEST)


def _enc_node_body(x_ref, w1_ref, b1_ref, w2_ref, b2_ref, g_ref, beta_ref,
                   o_ref):
    h = _relu(_fdot(x_ref[...], w1_ref[...]) + b1_ref[...])
    h = _relu(_fdot(h, w2_ref[...]) + b2_ref[...])
    o_ref[...] = _ln(h, g_ref[...], beta_ref[...])


def _enc_edge_body(x_ref, w1_ref, b1_ref, w2_ref, b2_ref, g_ref, beta_ref,
                   o_ref):
    h = _relu(_bdot(x_ref[...], w1_ref[...]) + b1_ref[...])
    h = _relu(_bdot(h, w2_ref[...]) + b2_ref[...])
    o_ref[...] = _ln(h, g_ref[...], beta_ref[...])


def _edge_body(a_ref, b_ref, ne_ref, ws_ref, wd_ref, w1c_ref, b1_ref,
               w2_ref, b2_ref, g_ref, beta_ref, msg_ref, neo_ref):
    a = a_ref[...]
    b = b_ref[...]
    ne = ne_ref[...]
    s = _bdot(a + b, ws_ref[...])
    dd = _bdot(a - b, wd_ref[...])
    pc = _bdot(ne, w1c_ref[...]) + b1_ref[...]
    u = 0.5 * (s + dd) + pc
    v = 0.5 * (s - dd) + pc
    w2 = w2_ref[...]
    b2 = b2_ref[...]
    g = g_ref[...]
    beta = beta_ref[...]
    hu = _relu(_bdot(_relu(u), w2) + b2)
    hv = _relu(_bdot(_relu(v), w2) + b2)
    msg_ref[...] = _ln(hu, g, beta)
    neo_ref[...] = ne + _ln(hv, g, beta)


def _node_body(p0_ref, p1_ref, nx_ref, w1a_ref, w1b_ref, b1_ref,
               w2_ref, b2_ref, g_ref, beta_ref, nxo_ref):
    aggr = p0_ref[0] + p1_ref[0]
    nx = nx_ref[...]
    u = _fdot(aggr, w1a_ref[...]) + _fdot(nx, w1b_ref[...]) + b1_ref[...]
    h = _relu(_fdot(_relu(u), w2_ref[...]) + b2_ref[...])
    nxo_ref[...] = nx + _ln(h, g_ref[...], beta_ref[...])


def _gru(xv, wr, wz, wn, cr, cz, cn, hn):
    r = jax.nn.sigmoid(_fdot(xv, wr) + cr)
    z = jax.nn.sigmoid(_fdot(xv, wz) + cz)
    n = jnp.tanh(_fdot(xv, wn) + cn + r * hn)
    return (1.0 - z) * n


def _decode_body(nx_ref, s_ref,
                 wr1_ref, wz1_ref, wn1_ref, cr1_ref, cz1_ref, cn1_ref, hn1_ref,
                 wr2_ref, wz2_ref, wn2_ref, cr2_ref, cz2_ref, cn2_ref, hn2_ref,
                 dw1a_ref, dw1b_ref, dw1c_ref, db1_ref, w2p_ref, b2p_ref,
                 o_ref):
    nx = nx_ref[...]
    h1 = _gru(nx, wr1_ref[...], wz1_ref[...], wn1_ref[...],
              cr1_ref[...], cz1_ref[...], cn1_ref[...], hn1_ref[...])
    h2 = _gru(h1, wr2_ref[...], wz2_ref[...], wn2_ref[...],
              cr2_ref[...], cz2_ref[...], cn2_ref[...], hn2_ref[...])
    hh = _relu(_fdot(h1, dw1a_ref[...]) + _fdot(h2, dw1b_ref[...]) +
               _fdot(s_ref[...], dw1c_ref[...]) + db1_ref[...])
    o_ref[...] = _fdot(hh, w2p_ref[...]) + b2p_ref[...]


def _full_spec(shape):
    return pl.BlockSpec(shape, lambda i: (0,) * len(shape))


def _row_spec(blk, width):
    return pl.BlockSpec((blk, width), lambda i: (i, 0))


def _run_enc(body, xp, p, rows, blk, fin, out_rows=None):
    grid = rows // blk
    return pl.pallas_call(
        body,
        grid=(grid,),
        in_specs=[
            _row_spec(blk, fin),
            _full_spec((fin, LATENT)), _full_spec((1, LATENT)),
            _full_spec((LATENT, LATENT)), _full_spec((1, LATENT)),
            _full_spec((1, LATENT)), _full_spec((1, LATENT)),
        ],
        out_specs=_row_spec(blk, LATENT),
        out_shape=jax.ShapeDtypeStruct((out_rows or rows, LATENT),
                                       jnp.float32),
    )(xp, p["W1"], p["b1"].reshape(1, -1), p["W2"], p["b2"].reshape(1, -1),
      p["g"].reshape(1, -1), p["beta"].reshape(1, -1))


def _run_edge(gathered, ne, p):
    w1a = p["W1"][:LATENT]
    w1b = p["W1"][LATENT:2 * LATENT]
    ws = w1a + w1b
    wd = w1a - w1b
    w1c = p["W1"][2 * LATENT:]
    return pl.pallas_call(
        _edge_body,
        grid=(GRID_E,),
        in_specs=[
            pl.BlockSpec((BLK_E, LATENT), lambda i: (2 * i, 0)),
            pl.BlockSpec((BLK_E, LATENT), lambda i: (2 * i + 1, 0)),
            _row_spec(BLK_E, LATENT),
            _full_spec((LATENT, LATENT)), _full_spec((LATENT, LATENT)),
            _full_spec((LATENT, LATENT)), _full_spec((1, LATENT)),
            _full_spec((LATENT, LATENT)), _full_spec((1, LATENT)),
            _full_spec((1, LATENT)), _full_spec((1, LATENT)),
        ],
        out_specs=[_row_spec(BLK_E, LATENT), _row_spec(BLK_E, LATENT)],
        out_shape=[jax.ShapeDtypeStruct((E_PAD, LATENT), jnp.float32),
                   jax.ShapeDtypeStruct((E_PAD, LATENT), jnp.float32)],
    )(gathered, gathered, ne, ws, wd, w1c, p["b1"].reshape(1, -1),
      p["W2"], p["b2"].reshape(1, -1), p["g"].reshape(1, -1),
      p["beta"].reshape(1, -1))


def _run_node(partials, nx, p):
    w1a = p["W1"][:LATENT]
    w1b = p["W1"][LATENT:]
    return pl.pallas_call(
        _node_body,
        grid=(GRID_N,),
        in_specs=[
            pl.BlockSpec((1, BLK_N, LATENT), lambda i: (0, i, 0)),
            pl.BlockSpec((1, BLK_N, LATENT), lambda i: (1, i, 0)),
            _row_spec(BLK_N, LATENT),
            _full_spec((LATENT, LATENT)), _full_spec((LATENT, LATENT)),
            _full_spec((1, LATENT)),
            _full_spec((LATENT, LATENT)), _full_spec((1, LATENT)),
            _full_spec((1, LATENT)), _full_spec((1, LATENT)),
        ],
        out_specs=_row_spec(BLK_N, LATENT),
        out_shape=jax.ShapeDtypeStruct((N, LATENT), jnp.float32),
    )(partials, partials, nx, w1a, w1b, p["b1"].reshape(1, -1),
      p["W2"], p["b2"].reshape(1, -1), p["g"].reshape(1, -1),
      p["beta"].reshape(1, -1))


def _gru_args(p):
    wr = p["Wih"][:, :LATENT]
    wz = p["Wih"][:, LATENT:2 * LATENT]
    wn = p["Wih"][:, 2 * LATENT:]
    cr = (p["bih"][:LATENT] + p["bhh"][:LATENT]).reshape(1, -1)
    cz = (p["bih"][LATENT:2 * LATENT] + p["bhh"][LATENT:2 * LATENT]).reshape(1, -1)
    cn = p["bih"][2 * LATENT:].reshape(1, -1)
    hn = p["bhh"][2 * LATENT:].reshape(1, -1)
    return wr, wz, wn, cr, cz, cn, hn


def _run_decode(nx, s, params):
    g1 = _gru_args(params["gru1"])
    g2 = _gru_args(params["gru2"])
    dec = params["dec"]
    dw1a = dec["W1"][:LATENT]
    dw1b = dec["W1"][LATENT:2 * LATENT]
    dw1c = dec["W1"][2 * LATENT:]
    out_dim = dec["W2"].shape[1]
    w2p = jnp.zeros((LATENT, 128), jnp.float32).at[:, :out_dim].set(dec["W2"])
    b2p = jnp.zeros((1, 128), jnp.float32).at[:, :out_dim].set(dec["b2"])
    gru_specs = [_full_spec((LATENT, LATENT))] * 3 + [_full_spec((1, LATENT))] * 4
    out_pad = pl.pallas_call(
        _decode_body,
        grid=(GRID_N,),
        in_specs=[_row_spec(BLK_N, LATENT), _row_spec(BLK_N, LATENT)]
                 + gru_specs + gru_specs
                 + [_full_spec((LATENT, LATENT))] * 3
                 + [_full_spec((1, LATENT)),
                    _full_spec((LATENT, 128)), _full_spec((1, 128))],
        out_specs=_row_spec(BLK_N, 128),
        out_shape=jax.ShapeDtypeStruct((N, 128), jnp.float32),
    )(nx, s, *g1, *g2, dw1a, dw1b, dw1c, dec["b1"].reshape(1, -1), w2p, b2p)
    return out_pad[:, :out_dim]


@functools.cache
def _sc_kernels():
    mesh = plsc.VectorSubcoreMesh(core_axis_name="c", subcore_axis_name="s",
                                  num_cores=NC, num_subcores=NS)

    @functools.partial(
        pl.kernel,
        out_type=jax.ShapeDtypeStruct((2 * E_PAD, LATENT), jnp.float32),
        mesh=mesh,
        scratch_types=[
            pltpu.VMEM((_G_NCH0, CH), jnp.int32),
            pltpu.VMEM((_G_ROWS, LATENT), jnp.float32),
            pltpu.VMEM((_G_ROWS, LATENT), jnp.float32),
            pltpu.SemaphoreType.DMA((2,)),
            pltpu.SemaphoreType.DMA((2,)),
        ],
        compiler_params=pltpu.CompilerParams(use_tc_tiling_on_sc=False),
    )
    def gather_k(table_hbm, idx_hbm, out_hbm, idx_v, rows0, rows1, gsem, wsem):
        cid = lax.axis_index("c")
        sid = lax.axis_index("s")
        nch = jnp.where(cid == 0, _G_NCH0, _G_NCH1)
        chunk0 = jnp.where(cid == 0, sid * _G_NCH0,
                           NS * _G_NCH0 + sid * _G_NCH1)
        ngrp = nch // _G_GRP
        pltpu.sync_copy(idx_hbm.at[pl.ds(chunk0, _G_NCH0)], idx_v)
        base = chunk0 * CH
        rows = (rows0, rows1)

        def fire(g, buf):
            for q in range(_G_GRP):
                pltpu.async_copy(table_hbm.at[idx_v.at[g * _G_GRP + q]],
                                 rows[buf].at[pl.ds(q * CH, CH)], gsem.at[buf])

        fire(0, 0)

        @pl.loop(0, ngrp, step=2)
        def _grp(g0):
            for p in range(2):
                g = g0 + p
                pltpu.make_async_copy(table_hbm.at[pl.ds(0, _G_ROWS)],
                                      rows[p], gsem.at[p]).wait()

                @pl.when(g >= 1)
                def _():
                    pltpu.make_async_copy(rows[1 - p],
                                          out_hbm.at[pl.ds(0, _G_ROWS)],
                                          wsem.at[1 - p]).wait()

                @pl.when(g + 1 < ngrp)
                def _():
                    fire(g + 1, 1 - p)

                pltpu.async_copy(rows[p],
                                 out_hbm.at[pl.ds(base + g * _G_ROWS, _G_ROWS)],
                                 wsem.at[p])

        pltpu.make_async_copy(rows[1], out_hbm.at[pl.ds(0, _G_ROWS)],
                              wsem.at[1]).wait()

    @functools.partial(
        pl.kernel,
        out_type=jax.ShapeDtypeStruct((NC, ACC_ROWS, LATENT), jnp.float32),
        mesh=mesh,
        scratch_types=[
            pltpu.VMEM((_S_NCH, CH), jnp.int32),
            pltpu.VMEM((_S_ROWS, LATENT), jnp.float32),
            pltpu.VMEM((_S_ROWS, LATENT), jnp.float32),
            pltpu.VMEM_SHARED((ACC_ROWS, LATENT), jnp.float32),
            pltpu.SemaphoreType.DMA((2,)),
        ],
        compiler_params=pltpu.CompilerParams(use_tc_tiling_on_sc=False),
    )
    def scatter_k(msg_hbm, idx_hbm, zeros_hbm, out_hbm, idx_v, msg0, msg1,
                  acc_sh, lsem):
        cid = lax.axis_index("c")
        sid = lax.axis_index("s")
        wid = sid * NC + cid
        pltpu.sync_copy(zeros_hbm.at[pl.ds(sid * ROWS_PER_TILE, ROWS_PER_TILE)],
                        acc_sh.at[pl.ds(sid * ROWS_PER_TILE, ROWS_PER_TILE)])
        pltpu.sync_copy(idx_hbm.at[wid], idx_v)
        plsc.subcore_barrier()
        base = wid * _S_PER_W
        msgb = (msg0, msg1)

        def fire(g, buf):
            pltpu.async_copy(msg_hbm.at[pl.ds(base + g * _S_ROWS, _S_ROWS)],
                             msgb[buf], lsem.at[buf])

        fire(0, 0)

        @pl.loop(0, _S_NGRP, step=2)
        def _grp(g0):
            for p in range(2):
                g = g0 + p
                pltpu.make_async_copy(msg_hbm.at[pl.ds(0, _S_ROWS)],
                                      msgb[p], lsem.at[p]).wait()

                @pl.when(g + 1 < _S_NGRP)
                def _():
                    fire(g + 1, 1 - p)

                for q in range(_S_GRP):
                    pltpu.sync_copy(msgb[p].at[pl.ds(q * CH, CH)],
                                    acc_sh.at[idx_v.at[g * _S_GRP + q]],
                                    add=True)

        plsc.subcore_barrier()
        pltpu.sync_copy(acc_sh.at[pl.ds(sid * ROWS_PER_TILE, ROWS_PER_TILE)],
                        out_hbm.at[cid, pl.ds(sid * ROWS_PER_TILE, ROWS_PER_TILE)])

    return gather_k, scatter_k


def _sc_gather(table, gidx):
    return _sc_kernels()[0](table, gidx)


def _sc_scatter(msg, sidx, zeros_acc):
    return _sc_kernels()[1](msg, sidx, zeros_acc)


def kernel(x, edge_attr, params, edge_index):
    row = edge_index[0].astype(jnp.int32)
    col = edge_index[1].astype(jnp.int32)
    pad = E_PAD - E
    zero_idx = jnp.zeros((pad,), jnp.int32)
    slab_pad = jnp.zeros((_G_NCH0, CH), jnp.int32)
    col_b = jnp.concatenate([col, zero_idx]).reshape(GRID_E, BLK_E // CH, CH)
    row_b = jnp.concatenate([row, zero_idx]).reshape(GRID_E, BLK_E // CH, CH)
    gidx = jnp.concatenate(
        [jnp.stack([col_b, row_b], axis=1).reshape(_G_TOT_CH, CH), slab_pad])
    sidx = jnp.concatenate([col, jnp.full((pad,), N, jnp.int32)]).reshape(
        NW, _S_NCH, CH)
    zeros_acc = jnp.zeros((ACC_ROWS, LATENT), jnp.float32)

    node_lat = _run_enc(_enc_node_body, x, params["node_enc"], N, BLK_N,
                        x.shape[1])
    edge_lat = _run_enc(_enc_edge_body, edge_attr, params["edge_enc"], E,
                        BLK_EE, edge_attr.shape[1], out_rows=E_PAD)

    nx = node_lat
    ne = edge_lat
    for _ in range(STEPS):
        gathered = _sc_gather(nx, gidx)
        msg, ne = _run_edge(gathered, ne, params["edge_net"])
        partials = _sc_scatter(msg, sidx, zeros_acc)
        nx = _run_node(partials, nx, params["node_net"])

    return _run_decode(nx, node_lat, params)


# speedup vs baseline: 1.0793x; 1.0793x over previous
import functools

import jax
import jax.numpy as jnp
from jax import lax
from jax.experimental import pallas as pl
from jax.experimental.pallas import tpu as pltpu
from jax.experimental.pallas import tpu_sc as plsc

N = 10000
E = 160000
LATENT = 64
STEPS = 3

NC = 2
NS = 16
NW = NC * NS
CH = 128
E_PAD = ((E + NW * CH - 1) // (NW * CH)) * (NW * CH)
BLK_E = 2048
GRID_E = E_PAD // BLK_E
PBLK_E = BLK_E // 2
BLK_EE = 1000
BLK_N = 2000
GRID_N = N // BLK_N
ACC_ROWS = ((N + NS - 1) // NS + 1) * NS
ROWS_PER_TILE = ACC_ROWS // NS
PROWS_PER_TILE = ROWS_PER_TILE // 2

_G_TOT_CH = 2 * E_PAD // CH
_G_NCH = _G_TOT_CH // NW
_G_GRP = 2
_G_NGRP = _G_NCH // _G_GRP
_G_ROWS = _G_GRP * CH
_S_PER_W = E_PAD // NW
_S_NCH = _S_PER_W // CH
_S_GRP = 2
_S_NGRP = _S_NCH // _S_GRP
_S_ROWS = _S_GRP * CH


def _bd2(w):
    a, b = w.shape
    z = jnp.zeros((2 * a, 2 * b), jnp.float32)
    return z.at[:a, :b].set(w).at[a:, b:].set(w)


def _t2(b):
    return jnp.tile(b.reshape(1, -1), (1, 2))


def _ln(h, g, beta):
    mu = jnp.mean(h, axis=-1, keepdims=True)
    var = jnp.mean((h - mu) ** 2, axis=-1, keepdims=True)
    return (h - mu) * jax.lax.rsqrt(var + 1e-5) * g + beta


def _ln2(h, g, beta):
    return jnp.concatenate(
        [_ln(h[:, :LATENT], g, beta), _ln(h[:, LATENT:], g, beta)], axis=1)


def _relu(v):
    return jnp.maximum(v, 0.0)


def _bdot(a, w):
    return jnp.dot(a.astype(jnp.bfloat16), w.astype(jnp.bfloat16),
                   preferred_element_type=jnp.float32)


def _fdot(a, w):
    return jnp.dot(a, w, preferred_element_type=jnp.float32,
                   precision=jax.lax.Precision.HIGHEST)


def _enc_node_body(x_ref, w1_ref, b1_ref, w2_ref, b2_ref, g_ref, beta_ref,
                   o_ref):
    h = _relu(_fdot(x_ref[...], w1_ref[...]) + b1_ref[...])
    h = _relu(_fdot(h, w2_ref[...]) + b2_ref[...])
    o_ref[...] = _ln(h, g_ref[...], beta_ref[...])


def _enc_edge_body(x_ref, w1_ref, b1_ref, w2_ref, b2_ref, g_ref, beta_ref,
                   o_ref):
    h = _relu(_bdot(x_ref[...], w1_ref[...]) + b1_ref[...])
    h = _relu(_bdot(h, w2_ref[...]) + b2_ref[...])
    o_ref[...] = _ln2(h, g_ref[...], beta_ref[...])


def _edge_body(a_ref, b_ref, ne_ref, ws_ref, wd_ref, w1c_ref, b1_ref,
               w2_ref, b2_ref, g_ref, beta_ref, msg_ref, neo_ref):
    a = a_ref[...]
    b = b_ref[...]
    ne = ne_ref[...]
    s = _bdot(a + b, ws_ref[...])
    dd = _bdot(a - b, wd_ref[...])
    pc = _bdot(ne, w1c_ref[...]) + b1_ref[...]
    u = 0.5 * (s + dd) + pc
    v = 0.5 * (s - dd) + pc
    w2 = w2_ref[...]
    b2 = b2_ref[...]
    g = g_ref[...]
    beta = beta_ref[...]
    hu = _relu(_bdot(_relu(u), w2) + b2)
    hv = _relu(_bdot(_relu(v), w2) + b2)
    msg_ref[...] = _ln2(hu, g, beta)
    neo_ref[...] = ne + _ln2(hv, g, beta)


def _node_body(p0_ref, p1_ref, nx_ref, w1a_ref, w1b_ref, b1_ref,
               w2_ref, b2_ref, g_ref, beta_ref, nxo_ref):
    aggr = p0_ref[0] + p1_ref[0]
    nx = nx_ref[...]
    u = _fdot(aggr, w1a_ref[...]) + _fdot(nx, w1b_ref[...]) + b1_ref[...]
    h = _relu(_fdot(_relu(u), w2_ref[...]) + b2_ref[...])
    nxo_ref[...] = nx + _ln(h, g_ref[...], beta_ref[...])


def _gru(xv, wr, wz, wn, cr, cz, cn, hn):
    r = jax.nn.sigmoid(_fdot(xv, wr) + cr)
    z = jax.nn.sigmoid(_fdot(xv, wz) + cz)
    n = jnp.tanh(_fdot(xv, wn) + cn + r * hn)
    return (1.0 - z) * n


def _decode_body(nx_ref, s_ref,
                 wr1_ref, wz1_ref, wn1_ref, cr1_ref, cz1_ref, cn1_ref, hn1_ref,
                 wr2_ref, wz2_ref, wn2_ref, cr2_ref, cz2_ref, cn2_ref, hn2_ref,
                 dw1a_ref, dw1b_ref, dw1c_ref, db1_ref, w2p_ref, b2p_ref,
                 o_ref):
    nx = nx_ref[...]
    h1 = _gru(nx, wr1_ref[...], wz1_ref[...], wn1_ref[...],
              cr1_ref[...], cz1_ref[...], cn1_ref[...], hn1_ref[...])
    h2 = _gru(h1, wr2_ref[...], wz2_ref[...], wn2_ref[...],
              cr2_ref[...], cz2_ref[...], cn2_ref[...], hn2_ref[...])
    hh = _relu(_fdot(h1, dw1a_ref[...]) + _fdot(h2, dw1b_ref[...]) +
               _fdot(s_ref[...], dw1c_ref[...]) + db1_ref[...])
    o_ref[...] = _fdot(hh, w2p_ref[...]) + b2p_ref[...]


def _full_spec(shape):
    return pl.BlockSpec(shape, lambda i: (0,) * len(shape))


def _row_spec(blk, width):
    return pl.BlockSpec((blk, width), lambda i: (i, 0))


def _run_enc_node(x, p):
    return pl.pallas_call(
        _enc_node_body,
        grid=(GRID_N,),
        in_specs=[
            _row_spec(BLK_N, 128),
            _full_spec((128, LATENT)), _full_spec((1, LATENT)),
            _full_spec((LATENT, LATENT)), _full_spec((1, LATENT)),
            _full_spec((1, LATENT)), _full_spec((1, LATENT)),
        ],
        out_specs=_row_spec(BLK_N, LATENT),
        out_shape=jax.ShapeDtypeStruct((N, LATENT), jnp.float32),
    )(x, p["W1"], p["b1"].reshape(1, -1), p["W2"], p["b2"].reshape(1, -1),
      p["g"].reshape(1, -1), p["beta"].reshape(1, -1))


def _run_enc_edge(ea2, p):
    fin = ea2.shape[1]
    return pl.pallas_call(
        _enc_edge_body,
        grid=(E // 2 // BLK_EE,),
        in_specs=[
            _row_spec(BLK_EE, fin),
            _full_spec((fin, 128)), _full_spec((1, 128)),
            _full_spec((128, 128)), _full_spec((1, 128)),
            _full_spec((1, LATENT)), _full_spec((1, LATENT)),
        ],
        out_specs=_row_spec(BLK_EE, 128),
        out_shape=jax.ShapeDtypeStruct((E_PAD // 2, 128), jnp.float32),
    )(ea2, _bd2(p["W1"]), _t2(p["b1"]), _bd2(p["W2"]), _t2(p["b2"]),
      p["g"].reshape(1, -1), p["beta"].reshape(1, -1))


def _run_edge(gathered, ne2, p):
    w1a = p["W1"][:LATENT]
    w1b = p["W1"][LATENT:2 * LATENT]
    w1c = p["W1"][2 * LATENT:]
    return pl.pallas_call(
        _edge_body,
        grid=(GRID_E,),
        in_specs=[
            pl.BlockSpec((PBLK_E, 128), lambda i: (2 * i, 0)),
            pl.BlockSpec((PBLK_E, 128), lambda i: (2 * i + 1, 0)),
            _row_spec(PBLK_E, 128),
            _full_spec((128, 128)), _full_spec((128, 128)),
            _full_spec((128, 128)), _full_spec((1, 128)),
            _full_spec((128, 128)), _full_spec((1, 128)),
            _full_spec((1, LATENT)), _full_spec((1, LATENT)),
        ],
        out_specs=[_row_spec(PBLK_E, 128), _row_spec(PBLK_E, 128)],
        out_shape=[jax.ShapeDtypeStruct((E_PAD // 2, 128), jnp.float32),
                   jax.ShapeDtypeStruct((E_PAD // 2, 128), jnp.float32)],
    )(gathered, gathered, ne2, _bd2(w1a + w1b), _bd2(w1a - w1b), _bd2(w1c),
      _t2(p["b1"]), _bd2(p["W2"]), _t2(p["b2"]),
      p["g"].reshape(1, -1), p["beta"].reshape(1, -1))


def _run_node(partials2, nx, p):
    w1a = p["W1"][:LATENT]
    w1b = p["W1"][LATENT:]
    return pl.pallas_call(
        _node_body,
        grid=(GRID_N,),
        in_specs=[
            pl.BlockSpec((1, BLK_N, LATENT), lambda i: (0, i, 0)),
            pl.BlockSpec((1, BLK_N, LATENT), lambda i: (1, i, 0)),
            _row_spec(BLK_N, LATENT),
            _full_spec((LATENT, LATENT)), _full_spec((LATENT, LATENT)),
            _full_spec((1, LATENT)),
            _full_spec((LATENT, LATENT)), _full_spec((1, LATENT)),
            _full_spec((1, LATENT)), _full_spec((1, LATENT)),
        ],
        out_specs=_row_spec(BLK_N, LATENT),
        out_shape=jax.ShapeDtypeStruct((N, LATENT), jnp.float32),
    )(partials2, partials2, nx, w1a, w1b, p["b1"].reshape(1, -1),
      p["W2"], p["b2"].reshape(1, -1), p["g"].reshape(1, -1),
      p["beta"].reshape(1, -1))


def _gru_args(p):
    wr = p["Wih"][:, :LATENT]
    wz = p["Wih"][:, LATENT:2 * LATENT]
    wn = p["Wih"][:, 2 * LATENT:]
    cr = (p["bih"][:LATENT] + p["bhh"][:LATENT]).reshape(1, -1)
    cz = (p["bih"][LATENT:2 * LATENT] + p["bhh"][LATENT:2 * LATENT]).reshape(1, -1)
    cn = p["bih"][2 * LATENT:].reshape(1, -1)
    hn = p["bhh"][2 * LATENT:].reshape(1, -1)
    return wr, wz, wn, cr, cz, cn, hn


def _run_decode(nx, s, params):
    g1 = _gru_args(params["gru1"])
    g2 = _gru_args(params["gru2"])
    dec = params["dec"]
    dw1a = dec["W1"][:LATENT]
    dw1b = dec["W1"][LATENT:2 * LATENT]
    dw1c = dec["W1"][2 * LATENT:]
    out_dim = dec["W2"].shape[1]
    w2p = jnp.zeros((LATENT, 128), jnp.float32).at[:, :out_dim].set(dec["W2"])
    b2p = jnp.zeros((1, 128), jnp.float32).at[:, :out_dim].set(dec["b2"])
    gru_specs = [_full_spec((LATENT, LATENT))] * 3 + [_full_spec((1, LATENT))] * 4
    out_pad = pl.pallas_call(
        _decode_body,
        grid=(GRID_N,),
        in_specs=[_row_spec(BLK_N, LATENT), _row_spec(BLK_N, LATENT)]
                 + gru_specs + gru_specs
                 + [_full_spec((LATENT, LATENT))] * 3
                 + [_full_spec((1, LATENT)),
                    _full_spec((LATENT, 128)), _full_spec((1, 128))],
        out_specs=_row_spec(BLK_N, 128),
        out_shape=jax.ShapeDtypeStruct((N, 128), jnp.float32),
    )(nx, s, *g1, *g2, dw1a, dw1b, dw1c, dec["b1"].reshape(1, -1), w2p, b2p)
    return out_pad[:, :out_dim]


def _pack_repack(src64, dst128, nrows128):
    @pl.loop(0, nrows128)
    def _rows(i):
        for j in range(8):
            v = src64[2 * i + (j // 4), pl.ds((j % 4) * 16, 16)]
            dst128[i, pl.ds(j * 16, 16)] = v


def _unpack_repack(src128, dst64, nrows128):
    @pl.loop(0, nrows128)
    def _rows(i):
        for j in range(8):
            v = src128[i, pl.ds(j * 16, 16)]
            dst64[2 * i + (j // 4), pl.ds((j % 4) * 16, 16)] = v


@functools.cache
def _sc_kernels():
    mesh = plsc.VectorSubcoreMesh(core_axis_name="c", subcore_axis_name="s",
                                  num_cores=NC, num_subcores=NS)

    @functools.partial(
        pl.kernel,
        out_type=jax.ShapeDtypeStruct((E_PAD, 128), jnp.float32),
        mesh=mesh,
        scratch_types=[
            pltpu.VMEM((_G_NCH, CH), jnp.int32),
            pltpu.VMEM((_G_ROWS, LATENT), jnp.float32),
            pltpu.VMEM((_G_ROWS, LATENT), jnp.float32),
            pltpu.VMEM((_G_ROWS // 2, 128), jnp.float32),
            pltpu.VMEM((_G_ROWS // 2, 128), jnp.float32),
            pltpu.SemaphoreType.DMA((2,)),
            pltpu.SemaphoreType.DMA((2,)),
        ],
        compiler_params=pltpu.CompilerParams(use_tc_tiling_on_sc=False),
    )
    def gather_k(table_hbm, idx_hbm, out_hbm, idx_v, bo0, bo1, st0, st1,
                 gsem, wsem):
        wid = lax.axis_index("s") * NC + lax.axis_index("c")
        pltpu.sync_copy(idx_hbm.at[wid], idx_v)
        base = wid * _G_NCH * CH // 2
        bo = (bo0, bo1)
        st = (st0, st1)

        def fire(g, buf):
            for q in range(_G_GRP):
                pltpu.async_copy(table_hbm.at[idx_v.at[g * _G_GRP + q]],
                                 bo[buf].at[pl.ds(q * CH, CH)], gsem.at[buf])

        fire(0, 0)

        @pl.loop(0, _G_NGRP, step=2)
        def _grp(g0):
            for p in range(2):
                g = g0 + p
                pltpu.make_async_copy(table_hbm.at[pl.ds(0, _G_ROWS)],
                                      bo[p], gsem.at[p]).wait()

                @pl.when(g + 1 < _G_NGRP)
                def _():
                    fire(g + 1, 1 - p)

                @pl.when(g >= 2)
                def _():
                    pltpu.make_async_copy(st[p],
                                          out_hbm.at[pl.ds(0, _G_ROWS // 2)],
                                          wsem.at[p]).wait()

                _pack_repack(bo[p], st[p], _G_ROWS // 2)
                pltpu.async_copy(
                    st[p],
                    out_hbm.at[pl.ds(base + g * (_G_ROWS // 2), _G_ROWS // 2)],
                    wsem.at[p])

        pltpu.make_async_copy(st[0], out_hbm.at[pl.ds(0, _G_ROWS // 2)],
                              wsem.at[0]).wait()
        pltpu.make_async_copy(st[1], out_hbm.at[pl.ds(0, _G_ROWS // 2)],
                              wsem.at[1]).wait()

    @functools.partial(
        pl.kernel,
        out_type=jax.ShapeDtypeStruct((NC, ACC_ROWS, LATENT), jnp.float32),
        mesh=mesh,
        scratch_types=[
            pltpu.VMEM((_S_NCH, CH), jnp.int32),
            pltpu.VMEM((_S_ROWS // 2, 128), jnp.float32),
            pltpu.VMEM((_S_ROWS // 2, 128), jnp.float32),
            pltpu.VMEM((_S_ROWS, LATENT), jnp.float32),
            pltpu.VMEM_SHARED((ACC_ROWS, LATENT), jnp.float32),
            pltpu.SemaphoreType.DMA((2,)),
        ],
        compiler_params=pltpu.CompilerParams(use_tc_tiling_on_sc=False),
    )
    def scatter_k(msg_hbm, idx_hbm, zeros_hbm, out_hbm, idx_v, st0, st1,
                  bounce, acc_sh, lsem):
        cid = lax.axis_index("c")
        sid = lax.axis_index("s")
        wid = sid * NC + cid
        pltpu.sync_copy(zeros_hbm.at[pl.ds(sid * ROWS_PER_TILE, ROWS_PER_TILE)],
                        acc_sh.at[pl.ds(sid * ROWS_PER_TILE, ROWS_PER_TILE)])
        pltpu.sync_copy(idx_hbm.at[wid], idx_v)
        plsc.subcore_barrier()
        base = wid * _S_PER_W // 2
        st = (st0, st1)

        def fire(g, buf):
            pltpu.async_copy(
                msg_hbm.at[pl.ds(base + g * (_S_ROWS // 2), _S_ROWS // 2)],
                st[buf], lsem.at[buf])

        fire(0, 0)

        @pl.loop(0, _S_NGRP, step=2)
        def _grp(g0):
            for p in range(2):
                g = g0 + p
                pltpu.make_async_copy(msg_hbm.at[pl.ds(0, _S_ROWS // 2)],
                                      st[p], lsem.at[p]).wait()

                @pl.when(g + 1 < _S_NGRP)
                def _():
                    fire(g + 1, 1 - p)

                _unpack_repack(st[p], bounce, _S_ROWS // 2)
                for q in range(_S_GRP):
                    pltpu.sync_copy(bounce.at[pl.ds(q * CH, CH)],
                                    acc_sh.at[idx_v.at[g * _S_GRP + q]],
                                    add=True)

        plsc.subcore_barrier()
        pltpu.sync_copy(acc_sh.at[pl.ds(sid * ROWS_PER_TILE, ROWS_PER_TILE)],
                        out_hbm.at[cid, pl.ds(sid * ROWS_PER_TILE,
                                              ROWS_PER_TILE)])

    return gather_k, scatter_k


def _sc_gather(table, gidx):
    return _sc_kernels()[0](table, gidx)


def _sc_scatter(msg2, sidx, zeros_acc):
    return _sc_kernels()[1](msg2, sidx, zeros_acc)


def kernel(x, edge_attr, params, edge_index):
    row = edge_index[0].astype(jnp.int32)
    col = edge_index[1].astype(jnp.int32)
    pad = E_PAD - E
    zero_idx = jnp.zeros((pad,), jnp.int32)
    col_b = jnp.concatenate([col, zero_idx]).reshape(GRID_E, BLK_E // CH, CH)
    row_b = jnp.concatenate([row, zero_idx]).reshape(GRID_E, BLK_E // CH, CH)
    gidx = jnp.stack([col_b, row_b], axis=1).reshape(NW, _G_NCH, CH)
    sidx = jnp.concatenate([col, jnp.full((pad,), N, jnp.int32)]).reshape(
        NW, _S_NCH, CH)
    zeros_acc = jnp.zeros((ACC_ROWS, LATENT), jnp.float32)

    node_lat = _run_enc_node(x, params["node_enc"])
    ea2 = edge_attr.reshape(E // 2, 2 * edge_attr.shape[1])
    ne2 = _run_enc_edge(ea2, params["edge_enc"])

    nx = node_lat
    for _ in range(STEPS):
        gathered = _sc_gather(nx, gidx)
        msg2, ne2 = _run_edge(gathered, ne2, params["edge_net"])
        partials2 = _sc_scatter(msg2, sidx, zeros_acc)
        nx = _run_node(partials2, nx, params["node_net"])

    return _run_decode(nx, node_lat, params)


# speedup vs baseline: 1.4022x; 1.2992x over previous
import functools

import jax
import jax.numpy as jnp
from jax import lax
from jax.experimental import pallas as pl
from jax.experimental.pallas import tpu as pltpu
from jax.experimental.pallas import tpu_sc as plsc

N = 10000
E = 160000
LATENT = 64
STEPS = 3

NC = 2
NS = 16
NW = NC * NS
CH = 128
E_PAD = ((E + NW * CH - 1) // (NW * CH)) * (NW * CH)
BLK_E = 2048
GRID_E = E_PAD // BLK_E
PBLK_E = BLK_E // 2
BLK_EE = 1000
BLK_N = 2000
GRID_N = N // BLK_N
ACC_ROWS = ((N + NS - 1) // NS + 1) * NS
ROWS_PER_TILE = ACC_ROWS // NS
PROWS_PER_TILE = ROWS_PER_TILE // 2

_G_TOT_CH = 2 * E_PAD // CH
_G_NCH = _G_TOT_CH // NW
_G_GRP = 2
_G_NGRP = _G_NCH // _G_GRP
_G_ROWS = _G_GRP * CH
_S_PER_W = E_PAD // NW
_S_NCH = _S_PER_W // CH
_S_GRP = 2
_S_NGRP = _S_NCH // _S_GRP
_S_ROWS = _S_GRP * CH


def _bd2(w):
    a, b = w.shape
    z = jnp.zeros((2 * a, 2 * b), jnp.float32)
    return z.at[:a, :b].set(w).at[a:, b:].set(w)


def _t2(b):
    return jnp.tile(b.reshape(1, -1), (1, 2))


def _m64():
    return _bd2(jnp.full((LATENT, LATENT), 1.0 / LATENT, jnp.float32))


def _ln(h, g, beta):
    mu = jnp.mean(h, axis=-1, keepdims=True)
    var = jnp.mean((h - mu) ** 2, axis=-1, keepdims=True)
    return (h - mu) * jax.lax.rsqrt(var + 1e-5) * g + beta


def _ln2(h, m_ref, g2, beta2):
    m = m_ref[...]
    mu = jnp.dot(h, m, preferred_element_type=jnp.float32)
    ex2 = jnp.dot(h * h, m, preferred_element_type=jnp.float32)
    var = ex2 - mu * mu
    return (h - mu) * jax.lax.rsqrt(var + 1e-5) * g2 + beta2


def _relu(v):
    return jnp.maximum(v, 0.0)


def _bdot(a, w):
    return jnp.dot(a.astype(jnp.bfloat16), w.astype(jnp.bfloat16),
                   preferred_element_type=jnp.float32)


def _fdot(a, w):
    return jnp.dot(a, w, preferred_element_type=jnp.float32,
                   precision=jax.lax.Precision.HIGHEST)


def _enc_node_body(x_ref, w1_ref, b1_ref, w2_ref, b2_ref, g_ref, beta_ref,
                   o_ref):
    h = _relu(_fdot(x_ref[...], w1_ref[...]) + b1_ref[...])
    h = _relu(_fdot(h, w2_ref[...]) + b2_ref[...])
    o_ref[...] = _ln(h, g_ref[...], beta_ref[...])


def _enc_edge_body(x_ref, w1_ref, b1_ref, w2_ref, b2_ref, m_ref, g_ref,
                   beta_ref, o_ref):
    h = _relu(_bdot(x_ref[...], w1_ref[...]) + b1_ref[...])
    h = _relu(_bdot(h, w2_ref[...]) + b2_ref[...])
    o_ref[...] = _ln2(h, m_ref, g_ref[...], beta_ref[...])


def _edge_body(a_ref, b_ref, ne_ref, ws_ref, wd_ref, w1c_ref, b1_ref,
               w2_ref, b2_ref, m_ref, g_ref, beta_ref, msg_ref, neo_ref):
    a = a_ref[...]
    b = b_ref[...]
    ne = ne_ref[...]
    s = _bdot(a + b, ws_ref[...])
    dd = _bdot(a - b, wd_ref[...])
    pc = _bdot(ne, w1c_ref[...]) + b1_ref[...]
    u = 0.5 * (s + dd) + pc
    v = 0.5 * (s - dd) + pc
    w2 = w2_ref[...]
    b2 = b2_ref[...]
    g = g_ref[...]
    beta = beta_ref[...]
    hu = _relu(_bdot(_relu(u), w2) + b2)
    hv = _relu(_bdot(_relu(v), w2) + b2)
    msg_ref[...] = _ln2(hu, m_ref, g, beta)
    neo_ref[...] = ne + _ln2(hv, m_ref, g, beta)


def _node_body(p0_ref, p1_ref, nx_ref, w1a_ref, w1b_ref, b1_ref,
               w2_ref, b2_ref, g_ref, beta_ref, nxo_ref):
    aggr = p0_ref[0] + p1_ref[0]
    nx = nx_ref[...]
    u = _fdot(aggr, w1a_ref[...]) + _fdot(nx, w1b_ref[...]) + b1_ref[...]
    h = _relu(_fdot(_relu(u), w2_ref[...]) + b2_ref[...])
    nxo_ref[...] = nx + _ln(h, g_ref[...], beta_ref[...])


def _gru(xv, wr, wz, wn, cr, cz, cn, hn):
    r = jax.nn.sigmoid(_fdot(xv, wr) + cr)
    z = jax.nn.sigmoid(_fdot(xv, wz) + cz)
    n = jnp.tanh(_fdot(xv, wn) + cn + r * hn)
    return (1.0 - z) * n


def _decode_body(nx_ref, s_ref,
                 wr1_ref, wz1_ref, wn1_ref, cr1_ref, cz1_ref, cn1_ref, hn1_ref,
                 wr2_ref, wz2_ref, wn2_ref, cr2_ref, cz2_ref, cn2_ref, hn2_ref,
                 dw1a_ref, dw1b_ref, dw1c_ref, db1_ref, w2p_ref, b2p_ref,
                 o_ref):
    nx = nx_ref[...]
    h1 = _gru(nx, wr1_ref[...], wz1_ref[...], wn1_ref[...],
              cr1_ref[...], cz1_ref[...], cn1_ref[...], hn1_ref[...])
    h2 = _gru(h1, wr2_ref[...], wz2_ref[...], wn2_ref[...],
              cr2_ref[...], cz2_ref[...], cn2_ref[...], hn2_ref[...])
    hh = _relu(_fdot(h1, dw1a_ref[...]) + _fdot(h2, dw1b_ref[...]) +
               _fdot(s_ref[...], dw1c_ref[...]) + db1_ref[...])
    o_ref[...] = _fdot(hh, w2p_ref[...]) + b2p_ref[...]


def _full_spec(shape):
    return pl.BlockSpec(shape, lambda i: (0,) * len(shape))


def _row_spec(blk, width):
    return pl.BlockSpec((blk, width), lambda i: (i, 0))


def _run_enc_node(x, p):
    return pl.pallas_call(
        _enc_node_body,
        grid=(GRID_N,),
        in_specs=[
            _row_spec(BLK_N, 128),
            _full_spec((128, LATENT)), _full_spec((1, LATENT)),
            _full_spec((LATENT, LATENT)), _full_spec((1, LATENT)),
            _full_spec((1, LATENT)), _full_spec((1, LATENT)),
        ],
        out_specs=_row_spec(BLK_N, LATENT),
        out_shape=jax.ShapeDtypeStruct((N, LATENT), jnp.float32),
    )(x, p["W1"], p["b1"].reshape(1, -1), p["W2"], p["b2"].reshape(1, -1),
      p["g"].reshape(1, -1), p["beta"].reshape(1, -1))


def _run_enc_edge(ea2, p):
    fin = ea2.shape[1]
    return pl.pallas_call(
        _enc_edge_body,
        grid=(E // 2 // BLK_EE,),
        in_specs=[
            _row_spec(BLK_EE, fin),
            _full_spec((fin, 128)), _full_spec((1, 128)),
            _full_spec((128, 128)), _full_spec((1, 128)),
            _full_spec((128, 128)),
            _full_spec((1, 128)), _full_spec((1, 128)),
        ],
        out_specs=_row_spec(BLK_EE, 128),
        out_shape=jax.ShapeDtypeStruct((E_PAD // 2, 128), jnp.float32),
    )(ea2, _bd2(p["W1"]), _t2(p["b1"]), _bd2(p["W2"]), _t2(p["b2"]),
      _m64(), _t2(p["g"]), _t2(p["beta"]))


def _run_edge(gathered, ne2, p):
    w1a = p["W1"][:LATENT]
    w1b = p["W1"][LATENT:2 * LATENT]
    w1c = p["W1"][2 * LATENT:]
    return pl.pallas_call(
        _edge_body,
        grid=(GRID_E,),
        in_specs=[
            pl.BlockSpec((PBLK_E, 128), lambda i: (2 * i, 0)),
            pl.BlockSpec((PBLK_E, 128), lambda i: (2 * i + 1, 0)),
            _row_spec(PBLK_E, 128),
            _full_spec((128, 128)), _full_spec((128, 128)),
            _full_spec((128, 128)), _full_spec((1, 128)),
            _full_spec((128, 128)), _full_spec((1, 128)),
            _full_spec((128, 128)),
            _full_spec((1, 128)), _full_spec((1, 128)),
        ],
        out_specs=[_row_spec(PBLK_E, 128), _row_spec(PBLK_E, 128)],
        out_shape=[jax.ShapeDtypeStruct((E_PAD // 2, 128), jnp.float32),
                   jax.ShapeDtypeStruct((E_PAD // 2, 128), jnp.float32)],
    )(gathered, gathered, ne2, _bd2(w1a + w1b), _bd2(w1a - w1b), _bd2(w1c),
      _t2(p["b1"]), _bd2(p["W2"]), _t2(p["b2"]),
      _m64(), _t2(p["g"]), _t2(p["beta"]))


def _run_node(partials2, nx, p):
    w1a = p["W1"][:LATENT]
    w1b = p["W1"][LATENT:]
    return pl.pallas_call(
        _node_body,
        grid=(GRID_N,),
        in_specs=[
            pl.BlockSpec((1, BLK_N, LATENT), lambda i: (0, i, 0)),
            pl.BlockSpec((1, BLK_N, LATENT), lambda i: (1, i, 0)),
            _row_spec(BLK_N, LATENT),
            _full_spec((LATENT, LATENT)), _full_spec((LATENT, LATENT)),
            _full_spec((1, LATENT)),
            _full_spec((LATENT, LATENT)), _full_spec((1, LATENT)),
            _full_spec((1, LATENT)), _full_spec((1, LATENT)),
        ],
        out_specs=_row_spec(BLK_N, LATENT),
        out_shape=jax.ShapeDtypeStruct((N, LATENT), jnp.float32),
    )(partials2, partials2, nx, w1a, w1b, p["b1"].reshape(1, -1),
      p["W2"], p["b2"].reshape(1, -1), p["g"].reshape(1, -1),
      p["beta"].reshape(1, -1))


def _gru_args(p):
    wr = p["Wih"][:, :LATENT]
    wz = p["Wih"][:, LATENT:2 * LATENT]
    wn = p["Wih"][:, 2 * LATENT:]
    cr = (p["bih"][:LATENT] + p["bhh"][:LATENT]).reshape(1, -1)
    cz = (p["bih"][LATENT:2 * LATENT] + p["bhh"][LATENT:2 * LATENT]).reshape(1, -1)
    cn = p["bih"][2 * LATENT:].reshape(1, -1)
    hn = p["bhh"][2 * LATENT:].reshape(1, -1)
    return wr, wz, wn, cr, cz, cn, hn


def _run_decode(nx, s, params):
    g1 = _gru_args(params["gru1"])
    g2 = _gru_args(params["gru2"])
    dec = params["dec"]
    dw1a = dec["W1"][:LATENT]
    dw1b = dec["W1"][LATENT:2 * LATENT]
    dw1c = dec["W1"][2 * LATENT:]
    out_dim = dec["W2"].shape[1]
    w2p = jnp.zeros((LATENT, 128), jnp.float32).at[:, :out_dim].set(dec["W2"])
    b2p = jnp.zeros((1, 128), jnp.float32).at[:, :out_dim].set(dec["b2"])
    gru_specs = [_full_spec((LATENT, LATENT))] * 3 + [_full_spec((1, LATENT))] * 4
    out_pad = pl.pallas_call(
        _decode_body,
        grid=(GRID_N,),
        in_specs=[_row_spec(BLK_N, LATENT), _row_spec(BLK_N, LATENT)]
                 + gru_specs + gru_specs
                 + [_full_spec((LATENT, LATENT))] * 3
                 + [_full_spec((1, LATENT)),
                    _full_spec((LATENT, 128)), _full_spec((1, 128))],
        out_specs=_row_spec(BLK_N, 128),
        out_shape=jax.ShapeDtypeStruct((N, 128), jnp.float32),
    )(nx, s, *g1, *g2, dw1a, dw1b, dw1c, dec["b1"].reshape(1, -1), w2p, b2p)
    return out_pad[:, :out_dim]


def _pack_repack(src64, dst128, nrows128):
    @pl.loop(0, nrows128)
    def _rows(i):
        for j in range(8):
            v = src64[2 * i + (j // 4), pl.ds((j % 4) * 16, 16)]
            dst128[i, pl.ds(j * 16, 16)] = v


def _unpack_repack(src128, dst64, nrows128):
    @pl.loop(0, nrows128)
    def _rows(i):
        for j in range(8):
            v = src128[i, pl.ds(j * 16, 16)]
            dst64[2 * i + (j // 4), pl.ds((j % 4) * 16, 16)] = v


@functools.cache
def _sc_kernels():
    mesh = plsc.VectorSubcoreMesh(core_axis_name="c", subcore_axis_name="s",
                                  num_cores=NC, num_subcores=NS)

    @functools.partial(
        pl.kernel,
        out_type=jax.ShapeDtypeStruct((E_PAD, 128), jnp.float32),
        mesh=mesh,
        scratch_types=[
            pltpu.VMEM((_G_NCH, CH), jnp.int32),
            pltpu.VMEM((_G_ROWS, LATENT), jnp.float32),
            pltpu.VMEM((_G_ROWS, LATENT), jnp.float32),
            pltpu.VMEM((_G_ROWS // 2, 128), jnp.float32),
            pltpu.VMEM((_G_ROWS // 2, 128), jnp.float32),
            pltpu.SemaphoreType.DMA((2,)),
            pltpu.SemaphoreType.DMA((2,)),
        ],
        compiler_params=pltpu.CompilerParams(use_tc_tiling_on_sc=False),
    )
    def gather_k(table_hbm, idx_hbm, out_hbm, idx_v, bo0, bo1, st0, st1,
                 gsem, wsem):
        wid = lax.axis_index("s") * NC + lax.axis_index("c")
        pltpu.sync_copy(idx_hbm.at[wid], idx_v)
        base = wid * _G_NCH * CH // 2
        bo = (bo0, bo1)
        st = (st0, st1)

        def fire(g, buf):
            for q in range(_G_GRP):
                pltpu.async_copy(table_hbm.at[idx_v.at[g * _G_GRP + q]],
                                 bo[buf].at[pl.ds(q * CH, CH)], gsem.at[buf])

        fire(0, 0)

        @pl.loop(0, _G_NGRP, step=2)
        def _grp(g0):
            for p in range(2):
                g = g0 + p
                pltpu.make_async_copy(table_hbm.at[pl.ds(0, _G_ROWS)],
                                      bo[p], gsem.at[p]).wait()

                @pl.when(g + 1 < _G_NGRP)
                def _():
                    fire(g + 1, 1 - p)

                @pl.when(g >= 2)
                def _():
                    pltpu.make_async_copy(st[p],
                                          out_hbm.at[pl.ds(0, _G_ROWS // 2)],
                                          wsem.at[p]).wait()

                _pack_repack(bo[p], st[p], _G_ROWS // 2)
                pltpu.async_copy(
                    st[p],
                    out_hbm.at[pl.ds(base + g * (_G_ROWS // 2), _G_ROWS // 2)],
                    wsem.at[p])

        pltpu.make_async_copy(st[0], out_hbm.at[pl.ds(0, _G_ROWS // 2)],
                              wsem.at[0]).wait()
        pltpu.make_async_copy(st[1], out_hbm.at[pl.ds(0, _G_ROWS // 2)],
                              wsem.at[1]).wait()

    @functools.partial(
        pl.kernel,
        out_type=jax.ShapeDtypeStruct((NC, ACC_ROWS, LATENT), jnp.float32),
        mesh=mesh,
        scratch_types=[
            pltpu.VMEM((_S_NCH, CH), jnp.int32),
            pltpu.VMEM((_S_ROWS // 2, 128), jnp.float32),
            pltpu.VMEM((_S_ROWS // 2, 128), jnp.float32),
            pltpu.VMEM((_S_ROWS, LATENT), jnp.float32),
            pltpu.VMEM_SHARED((ACC_ROWS, LATENT), jnp.float32),
            pltpu.SemaphoreType.DMA((2,)),
        ],
        compiler_params=pltpu.CompilerParams(use_tc_tiling_on_sc=False),
    )
    def scatter_k(msg_hbm, idx_hbm, zeros_hbm, out_hbm, idx_v, st0, st1,
                  bounce, acc_sh, lsem):
        cid = lax.axis_index("c")
        sid = lax.axis_index("s")
        wid = sid * NC + cid
        pltpu.sync_copy(zeros_hbm.at[pl.ds(sid * ROWS_PER_TILE, ROWS_PER_TILE)],
                        acc_sh.at[pl.ds(sid * ROWS_PER_TILE, ROWS_PER_TILE)])
        pltpu.sync_copy(idx_hbm.at[wid], idx_v)
        plsc.subcore_barrier()
        base = wid * _S_PER_W // 2
        st = (st0, st1)

        def fire(g, buf):
            pltpu.async_copy(
                msg_hbm.at[pl.ds(base + g * (_S_ROWS // 2), _S_ROWS // 2)],
                st[buf], lsem.at[buf])

        fire(0, 0)

        @pl.loop(0, _S_NGRP, step=2)
        def _grp(g0):
            for p in range(2):
                g = g0 + p
                pltpu.make_async_copy(msg_hbm.at[pl.ds(0, _S_ROWS // 2)],
                                      st[p], lsem.at[p]).wait()

                @pl.when(g + 1 < _S_NGRP)
                def _():
                    fire(g + 1, 1 - p)

                _unpack_repack(st[p], bounce, _S_ROWS // 2)
                for q in range(_S_GRP):
                    pltpu.sync_copy(bounce.at[pl.ds(q * CH, CH)],
                                    acc_sh.at[idx_v.at[g * _S_GRP + q]],
                                    add=True)

        plsc.subcore_barrier()
        pltpu.sync_copy(acc_sh.at[pl.ds(sid * ROWS_PER_TILE, ROWS_PER_TILE)],
                        out_hbm.at[cid, pl.ds(sid * ROWS_PER_TILE,
                                              ROWS_PER_TILE)])

    return gather_k, scatter_k


def _sc_gather(table, gidx):
    return _sc_kernels()[0](table, gidx)


def _sc_scatter(msg2, sidx, zeros_acc):
    return _sc_kernels()[1](msg2, sidx, zeros_acc)


def kernel(x, edge_attr, params, edge_index):
    row = edge_index[0].astype(jnp.int32)
    col = edge_index[1].astype(jnp.int32)
    pad = E_PAD - E
    zero_idx = jnp.zeros((pad,), jnp.int32)
    col_b = jnp.concatenate([col, zero_idx]).reshape(GRID_E, BLK_E // CH, CH)
    row_b = jnp.concatenate([row, zero_idx]).reshape(GRID_E, BLK_E // CH, CH)
    gidx = jnp.stack([col_b, row_b], axis=1).reshape(NW, _G_NCH, CH)
    sidx = jnp.concatenate([col, jnp.full((pad,), N, jnp.int32)]).reshape(
        NW, _S_NCH, CH)
    zeros_acc = jnp.zeros((ACC_ROWS, LATENT), jnp.float32)

    node_lat = _run_enc_node(x, params["node_enc"])
    ea2 = edge_attr.reshape(E // 2, 2 * edge_attr.shape[1])
    ne2 = _run_enc_edge(ea2, params["edge_enc"])

    nx = node_lat
    for _ in range(STEPS):
        gathered = _sc_gather(nx, gidx)
        msg2, ne2 = _run_edge(gathered, ne2, params["edge_net"])
        partials2 = _sc_scatter(msg2, sidx, zeros_acc)
        nx = _run_node(partials2, nx, params["node_net"])

    return _run_decode(nx, node_lat, params)
